# baseline (speedup 1.0000x reference)
import jax
import jax.numpy as jnp
from jax import lax
from jax.experimental import pallas as pl
from jax.experimental.pallas import tpu as pltpu, tpu_sc as plsc

_MAX_LEN = 8192
_DIM = 768
_BATCH = 2
_NW = 32
_ROWS_PER_W = _MAX_LEN // _NW
_CHUNK_ROWS = 32
_N_CHUNKS = _ROWS_PER_W // _CHUNK_ROWS
_N_ITEMS = _N_CHUNKS * _BATCH
_LANE_STEPS = _DIM // 16


def _sc_kernel(x_hbm, pos_hbm, out_hbm,
               x_v0, x_v1, x_v2, pos_v0, pos_v1,
               in_s0, in_s1, in_s2, out_s0, out_s1, out_s2,
               pos_s0, pos_s1):
    x_vs = [x_v0, x_v1, x_v2]
    pos_vs = [pos_v0, pos_v1]
    in_sems = [in_s0, in_s1, in_s2]
    out_sems = [out_s0, out_s1, out_s2]
    pos_sems = [pos_s0, pos_s1]

    wid = lax.axis_index("c") * 16 + lax.axis_index("s")
    row0 = wid * _ROWS_PER_W

    def x_in(k):
        ci, b, xb = k // 2, k % 2, k % 3
        pltpu.async_copy(
            x_hbm.at[b, pl.ds(row0 + ci * _CHUNK_ROWS, _CHUNK_ROWS), :],
            x_vs[xb], in_sems[xb])

    def pos_in(ci):
        pltpu.async_copy(
            pos_hbm.at[pl.ds(row0 + ci * _CHUNK_ROWS, _CHUNK_ROWS), :],
            pos_vs[ci % 2], pos_sems[ci % 2])

    pos_in(0)
    pos_in(1)
    x_in(0)
    x_in(1)

    for k in range(_N_ITEMS):
        ci, b, xb, pb = k // 2, k % 2, k % 3, (k // 2) % 2
        pltpu.make_async_copy(
            x_hbm.at[b, pl.ds(row0 + ci * _CHUNK_ROWS, _CHUNK_ROWS), :],
            x_vs[xb], in_sems[xb]).wait()
        if b == 0:
            pltpu.make_async_copy(
                pos_hbm.at[pl.ds(row0 + ci * _CHUNK_ROWS, _CHUNK_ROWS), :],
                pos_vs[pb], pos_sems[pb]).wait()

        def body(r, _, xb=xb, pb=pb):
            for j in range(_LANE_STEPS):
                o = j * 16
                x_vs[xb][r, pl.ds(o, 16)] = (x_vs[xb][r, pl.ds(o, 16)]
                                             + pos_vs[pb][r, pl.ds(o, 16)])
            return 0

        lax.fori_loop(0, _CHUNK_ROWS, body, 0)

        pltpu.async_copy(
            x_vs[xb],
            out_hbm.at[b, pl.ds(row0 + ci * _CHUNK_ROWS, _CHUNK_ROWS), :],
            out_sems[xb])

        if b == 1 and ci + 2 < _N_CHUNKS:
            pos_in(ci + 2)

        kn = k + 2
        if kn < _N_ITEMS:
            if kn >= 3:
                cp, bp = (kn - 3) // 2, (kn - 3) % 2
                pltpu.make_async_copy(
                    x_vs[kn % 3],
                    out_hbm.at[bp, pl.ds(row0 + cp * _CHUNK_ROWS,
                                         _CHUNK_ROWS), :],
                    out_sems[kn % 3]).wait()
            x_in(kn)

    for k in range(_N_ITEMS - 3, _N_ITEMS):
        ci, b = k // 2, k % 2
        pltpu.make_async_copy(
            x_vs[k % 3],
            out_hbm.at[b, pl.ds(row0 + ci * _CHUNK_ROWS, _CHUNK_ROWS), :],
            out_sems[k % 3]).wait()


def kernel(x, pos_table):
    batch, max_len, dim = x.shape
    mesh = plsc.VectorSubcoreMesh(core_axis_name="c", subcore_axis_name="s")
    return pl.kernel(
        _sc_kernel,
        mesh=mesh,
        out_type=jax.ShapeDtypeStruct((batch, max_len, dim), jnp.float32),
        scratch_types=(
            [pltpu.VMEM((_CHUNK_ROWS, _DIM), jnp.float32)] * 3
            + [pltpu.VMEM((_CHUNK_ROWS, _DIM), jnp.float32)] * 2
            + [pltpu.SemaphoreType.DMA] * 8
        ),
    )(x, pos_table)
